# baseline (speedup 1.0000x reference)
import functools

import jax
import jax.numpy as jnp
from jax import lax
from jax.experimental import pallas as pl
from jax.experimental.pallas import tpu as pltpu
from jax.experimental.pallas import tpu_sc as plsc

CH = 128
G = 4
ROWS_PER_GROUP = CH * G

VB = 512


@functools.cache
def _make_sc_gather(NW, NGRP, D):
    mesh = plsc.VectorSubcoreMesh(core_axis_name="c", subcore_axis_name="s")
    info = plsc.get_sparse_core_info()
    NC = info.num_cores

    @functools.partial(
        pl.kernel,
        mesh=mesh,
        compiler_params=pltpu.CompilerParams(use_tc_tiling_on_sc=False),
        out_type=jax.ShapeDtypeStruct((NW, NGRP, ROWS_PER_GROUP, D), jnp.float32),
        scratch_types=[
            pltpu.VMEM((NGRP, G, CH), jnp.int32),
            pltpu.VMEM((2, ROWS_PER_GROUP, D), jnp.float32),
            pltpu.SemaphoreType.DMA,
            pltpu.SemaphoreType.DMA,
        ],
    )
    def sc_gather(table_hbm, idx_hbm, out_hbm, idx_v, rows_v, sem0, sem1):
        wid = lax.axis_index("s") * NC + lax.axis_index("c")
        pltpu.sync_copy(idx_hbm.at[wid], idx_v)
        sems = (sem0, sem1)

        def fire(g, b):
            for j in range(G):
                pltpu.async_copy(
                    table_hbm.at[idx_v.at[g, j]],
                    rows_v.at[b, pl.ds(j * CH, CH)],
                    sems[b],
                )

        def drain(b):
            pltpu.make_async_copy(
                table_hbm.at[pl.ds(0, ROWS_PER_GROUP)], rows_v.at[b], sems[b]
            ).wait()

        fire(0, 0)
        fire(1, 1)

        def body(i, carry):
            for b in range(2):
                g = 2 * i + b
                drain(b)
                pltpu.sync_copy(rows_v.at[b], out_hbm.at[wid, g])
                fire(g + 2, b)
            return carry

        lax.fori_loop(0, NGRP // 2 - 1, body, 0)
        for b in range(2):
            g = NGRP - 2 + b
            drain(b)
            pltpu.sync_copy(rows_v.at[b], out_hbm.at[wid, g])

    return sc_gather


def _project_body(t1_ref, t2_ref, w_ref, b2_ref, o_ref):
    w = w_ref[...]
    cn = (((0,), (0,)), ((), ()))
    z1 = lax.dot_general(t1_ref[...], w, cn,
                         preferred_element_type=jnp.float32)
    z2 = lax.dot_general(t2_ref[...], w, cn,
                         preferred_element_type=jnp.float32)
    z = jnp.concatenate([z1, z2], axis=1)
    o_ref[...] = z + b2_ref[...]


def _project_table(table_t, Wt, b2):
    d_in, vocab = table_t.shape
    grid = pl.cdiv(vocab, 2 * VB)
    n2 = grid * VB
    return pl.pallas_call(
        _project_body,
        grid=(grid,),
        in_specs=[
            pl.BlockSpec((d_in, VB), lambda i: (0, 2 * i)),
            pl.BlockSpec((d_in, VB), lambda i: (0, 2 * i + 1)),
            pl.BlockSpec((d_in, d_in), lambda i: (0, 0)),
            pl.BlockSpec((1, 128), lambda i: (0, 0)),
        ],
        out_specs=pl.BlockSpec((VB, 128), lambda i: (i, 0)),
        out_shape=jax.ShapeDtypeStruct((n2, 128), jnp.float32),
    )(table_t, table_t, Wt, b2)


def _unpack_body(e_ref, g_ref, o_ref):
    t_all = lax.dot_general(
        e_ref[...], g_ref[0], (((0,), (1,)), ((), ())),
        preferred_element_type=jnp.float32,
    )
    half = pl.program_id(1) >= 16
    o_ref[0] = jnp.where(half, t_all[64:], t_all[:64])


def _transpose_out(g_packed, seq, batch, d_out):
    cb = 512
    half_blocks = (batch // 2) // cb
    return pl.pallas_call(
        _unpack_body,
        grid=(seq, 2 * half_blocks),
        in_specs=[
            pl.BlockSpec((128, 128), lambda s, j: (0, 0)),
            pl.BlockSpec((1, cb, 128),
                         lambda s, j: (s, lax.rem(j, half_blocks), 0)),
        ],
        out_specs=pl.BlockSpec((1, d_out, cb), lambda s, j: (s, 0, j)),
        out_shape=jax.ShapeDtypeStruct((seq, d_out, batch), jnp.float32),
    )(jnp.eye(128, dtype=jnp.float32), g_packed)


def kernel(indices, table, W, b):
    batch, seq = indices.shape
    vocab, d = table.shape
    d_out = W.shape[1]
    n_rows = batch * seq

    table_t = jnp.transpose(table)
    idx_t = jnp.transpose(indices.astype(jnp.int32))

    u2 = _project_table(table_t, W, jnp.tile(b, 2).reshape(1, 2 * d_out))
    u_rows = u2.reshape(u2.shape[0] * 2, d_out)

    t_tile = idx_t >> 9
    v_packed = (((t_tile >> 1) << 9) | (idx_t & (VB - 1))) * 2 + (t_tile & 1)
    idx_feed = (
        v_packed.reshape(seq, 2, batch // 2)
        .transpose(0, 2, 1)
        .reshape(seq, batch)
    )

    info = plsc.get_sparse_core_info()
    NW = info.num_cores * info.num_subcores
    per_w = n_rows // NW
    assert per_w * NW == n_rows and per_w % ROWS_PER_GROUP == 0
    ngrp = per_w // ROWS_PER_GROUP

    idx4 = idx_feed.reshape(NW, ngrp, G, CH)
    gathered = _make_sc_gather(NW, ngrp, d_out)(u_rows, idx4)

    g_packed = gathered.reshape(seq, batch // 2, 2 * d_out)
    p = _transpose_out(g_packed, seq, batch, d_out)
    return jnp.transpose(p, (2, 0, 1))


# speedup vs baseline: 1.9466x; 1.9466x over previous
import functools

import jax
import jax.numpy as jnp
from jax import lax
from jax.experimental import pallas as pl
from jax.experimental.pallas import tpu as pltpu
from jax.experimental.pallas import tpu_sc as plsc

CH = 128
G = 4
ROWS_PER_GROUP = CH * G

VB = 2048
VB_SHIFT = 11


@functools.cache
def _make_sc_gather(NW, NGRP, D):
    mesh = plsc.VectorSubcoreMesh(core_axis_name="c", subcore_axis_name="s")
    info = plsc.get_sparse_core_info()
    NC = info.num_cores

    @functools.partial(
        pl.kernel,
        mesh=mesh,
        compiler_params=pltpu.CompilerParams(use_tc_tiling_on_sc=False),
        out_type=jax.ShapeDtypeStruct((NW, NGRP, ROWS_PER_GROUP, D), jnp.float32),
        scratch_types=[
            pltpu.VMEM((NGRP, G, CH), jnp.int32),
            pltpu.VMEM((2, ROWS_PER_GROUP, D), jnp.float32),
            pltpu.SemaphoreType.DMA,
            pltpu.SemaphoreType.DMA,
        ],
    )
    def sc_gather(table_hbm, idx_hbm, out_hbm, idx_v, rows_v, sem0, sem1):
        wid = lax.axis_index("s") * NC + lax.axis_index("c")
        pltpu.sync_copy(idx_hbm.at[wid], idx_v)
        sems = (sem0, sem1)

        def fire(g, b):
            for j in range(G):
                pltpu.async_copy(
                    table_hbm.at[idx_v.at[g, j]],
                    rows_v.at[b, pl.ds(j * CH, CH)],
                    sems[b],
                )

        def drain(b):
            pltpu.make_async_copy(
                table_hbm.at[pl.ds(0, ROWS_PER_GROUP)], rows_v.at[b], sems[b]
            ).wait()

        fire(0, 0)
        fire(1, 1)

        def body(i, carry):
            for b in range(2):
                g = 2 * i + b
                drain(b)
                pltpu.sync_copy(rows_v.at[b], out_hbm.at[wid, g])
                fire(g + 2, b)
            return carry

        lax.fori_loop(0, NGRP // 2 - 1, body, 0)
        for b in range(2):
            g = NGRP - 2 + b
            drain(b)
            pltpu.sync_copy(rows_v.at[b], out_hbm.at[wid, g])

    return sc_gather


def _project_body(t1_ref, t2_ref, w_ref, b2_ref, o_ref):
    w = w_ref[...]
    cn = (((0,), (0,)), ((), ()))
    z1 = lax.dot_general(t1_ref[...], w, cn,
                         preferred_element_type=jnp.float32)
    z2 = lax.dot_general(t2_ref[...], w, cn,
                         preferred_element_type=jnp.float32)
    z = jnp.concatenate([z1, z2], axis=1)
    o_ref[...] = z + b2_ref[...]


def _project_table(table_t, Wt, b2):
    d_in, vocab = table_t.shape
    grid = pl.cdiv(vocab, 2 * VB)
    n2 = grid * VB
    last = pl.cdiv(vocab, VB) - 1
    return pl.pallas_call(
        _project_body,
        grid=(grid,),
        in_specs=[
            pl.BlockSpec((d_in, VB), lambda i: (0, jnp.minimum(2 * i, last))),
            pl.BlockSpec(
                (d_in, VB), lambda i: (0, jnp.minimum(2 * i + 1, last))
            ),
            pl.BlockSpec((d_in, d_in), lambda i: (0, 0)),
            pl.BlockSpec((1, 128), lambda i: (0, 0)),
        ],
        out_specs=pl.BlockSpec((VB, 128), lambda i: (i, 0)),
        out_shape=jax.ShapeDtypeStruct((n2, 128), jnp.float32),
    )(table_t, table_t, Wt, b2)


def _make_unpack_body(half_blocks):
    def _unpack_body(g_ref, o_ref):
        t_all = g_ref[0].T
        half = pl.program_id(1) >= half_blocks
        o_ref[0] = jnp.where(half, t_all[64:], t_all[:64])
    return _unpack_body


def _transpose_out(g_packed, seq, batch, d_out):
    cb = 2048
    half_blocks = (batch // 2) // cb
    return pl.pallas_call(
        _make_unpack_body(half_blocks),
        grid=(seq, 2 * half_blocks),
        in_specs=[
            pl.BlockSpec((1, cb, 128),
                         lambda s, j: (s, lax.rem(j, half_blocks), 0)),
        ],
        out_specs=pl.BlockSpec((1, d_out, cb), lambda s, j: (s, 0, j)),
        out_shape=jax.ShapeDtypeStruct((seq, d_out, batch), jnp.float32),
    )(g_packed)


def kernel(indices, table, W, b):
    batch, seq = indices.shape
    vocab, d = table.shape
    d_out = W.shape[1]
    n_rows = batch * seq

    table_t = jnp.transpose(table)
    idx_t = jnp.transpose(indices.astype(jnp.int32))

    u2 = _project_table(table_t, W, jnp.tile(b, 2).reshape(1, 2 * d_out))
    u_rows = u2.reshape(u2.shape[0] * 2, d_out)

    t_tile = idx_t >> VB_SHIFT
    v_packed = (
        (((t_tile >> 1) << VB_SHIFT) | (idx_t & (VB - 1))) * 2 + (t_tile & 1)
    )
    idx_feed = (
        v_packed.reshape(seq, 2, batch // 2)
        .transpose(0, 2, 1)
        .reshape(seq, batch)
    )

    info = plsc.get_sparse_core_info()
    NW = info.num_cores * info.num_subcores
    per_w = n_rows // NW
    assert per_w * NW == n_rows and per_w % ROWS_PER_GROUP == 0
    ngrp = per_w // ROWS_PER_GROUP

    idx4 = idx_feed.reshape(NW, ngrp, G, CH)
    gathered = _make_sc_gather(NW, ngrp, d_out)(u_rows, idx4)

    g_packed = gathered.reshape(seq, batch // 2, 2 * d_out)
    p = _transpose_out(g_packed, seq, batch, d_out)
    return jnp.transpose(p, (2, 0, 1))


# speedup vs baseline: 2.3059x; 1.1846x over previous
import functools

import jax
import jax.numpy as jnp
from jax import lax
from jax.experimental import pallas as pl
from jax.experimental.pallas import tpu as pltpu
from jax.experimental.pallas import tpu_sc as plsc

CH = 128
G = 4
ROWS_PER_GROUP = CH * G

VB = 4096
VB_SHIFT = 12


@functools.cache
def _make_sc_gather(NW, NGRP, D):
    mesh = plsc.VectorSubcoreMesh(core_axis_name="c", subcore_axis_name="s")
    info = plsc.get_sparse_core_info()
    NC = info.num_cores

    @functools.partial(
        pl.kernel,
        mesh=mesh,
        compiler_params=pltpu.CompilerParams(use_tc_tiling_on_sc=False),
        out_type=jax.ShapeDtypeStruct((NW, NGRP, ROWS_PER_GROUP, D), jnp.float32),
        scratch_types=[
            pltpu.VMEM((NGRP, G, CH), jnp.int32),
            pltpu.VMEM((2, ROWS_PER_GROUP, D), jnp.float32),
            pltpu.SemaphoreType.DMA,
            pltpu.SemaphoreType.DMA,
        ],
    )
    def sc_gather(table_hbm, idx_hbm, out_hbm, idx_v, rows_v, sem0, sem1):
        wid = lax.axis_index("s") * NC + lax.axis_index("c")
        pltpu.sync_copy(idx_hbm.at[wid], idx_v)
        sems = (sem0, sem1)

        def fire(g, b):
            for j in range(G):
                pltpu.async_copy(
                    table_hbm.at[idx_v.at[g, j]],
                    rows_v.at[b, pl.ds(j * CH, CH)],
                    sems[b],
                )

        def drain(b):
            pltpu.make_async_copy(
                table_hbm.at[pl.ds(0, ROWS_PER_GROUP)], rows_v.at[b], sems[b]
            ).wait()

        fire(0, 0)
        fire(1, 1)

        def body(i, carry):
            for b in range(2):
                g = 2 * i + b
                drain(b)
                pltpu.sync_copy(rows_v.at[b], out_hbm.at[wid, g])
                fire(g + 2, b)
            return carry

        lax.fori_loop(0, NGRP // 2 - 1, body, 0)
        for b in range(2):
            g = NGRP - 2 + b
            drain(b)
            pltpu.sync_copy(rows_v.at[b], out_hbm.at[wid, g])

    return sc_gather


def _project_body(t1_ref, t2_ref, w_ref, b2_ref, o_ref):
    w = w_ref[...]
    cn = (((0,), (0,)), ((), ()))
    z1 = lax.dot_general(t1_ref[...], w, cn,
                         preferred_element_type=jnp.float32)
    z2 = lax.dot_general(t2_ref[...], w, cn,
                         preferred_element_type=jnp.float32)
    z = jnp.concatenate([z1, z2], axis=1)
    o_ref[...] = z + b2_ref[...]


def _project_table(table_t, Wt, b2):
    d_in, vocab = table_t.shape
    grid = pl.cdiv(vocab, 2 * VB)
    n2 = grid * VB
    last = pl.cdiv(vocab, VB) - 1
    return pl.pallas_call(
        _project_body,
        grid=(grid,),
        in_specs=[
            pl.BlockSpec((d_in, VB), lambda i: (0, jnp.minimum(2 * i, last))),
            pl.BlockSpec(
                (d_in, VB), lambda i: (0, jnp.minimum(2 * i + 1, last))
            ),
            pl.BlockSpec((d_in, d_in), lambda i: (0, 0)),
            pl.BlockSpec((1, 128), lambda i: (0, 0)),
        ],
        out_specs=pl.BlockSpec((VB, 128), lambda i: (i, 0)),
        out_shape=jax.ShapeDtypeStruct((n2, 128), jnp.float32),
    )(table_t, table_t, Wt, b2)


def _make_unpack_body(half_blocks):
    def _unpack_body(g_ref, o_ref):
        t_all = g_ref[0].T
        half = pl.program_id(1) >= half_blocks
        o_ref[0] = jnp.where(half, t_all[64:], t_all[:64])
    return _unpack_body


def _transpose_out(g_packed, seq, batch, d_out):
    cb = 4096
    half_blocks = (batch // 2) // cb
    return pl.pallas_call(
        _make_unpack_body(half_blocks),
        grid=(seq, 2 * half_blocks),
        in_specs=[
            pl.BlockSpec((1, cb, 128),
                         lambda s, j: (s, lax.rem(j, half_blocks), 0)),
        ],
        out_specs=pl.BlockSpec((1, d_out, cb), lambda s, j: (s, 0, j)),
        out_shape=jax.ShapeDtypeStruct((seq, d_out, batch), jnp.float32),
    )(g_packed)


def kernel(indices, table, W, b):
    batch, seq = indices.shape
    vocab, d = table.shape
    d_out = W.shape[1]
    n_rows = batch * seq

    table_t = jnp.transpose(table)
    idx_t = jnp.transpose(indices.astype(jnp.int32))

    u2 = _project_table(table_t, W, jnp.tile(b, 2).reshape(1, 2 * d_out))
    u_rows = u2.reshape(u2.shape[0] * 2, d_out)

    t_tile = idx_t >> VB_SHIFT
    v_packed = (
        (((t_tile >> 1) << VB_SHIFT) | (idx_t & (VB - 1))) * 2 + (t_tile & 1)
    )
    idx_feed = (
        v_packed.reshape(seq, 2, batch // 2)
        .transpose(0, 2, 1)
        .reshape(seq, batch)
    )

    info = plsc.get_sparse_core_info()
    NW = info.num_cores * info.num_subcores
    per_w = n_rows // NW
    assert per_w * NW == n_rows and per_w % ROWS_PER_GROUP == 0
    ngrp = per_w // ROWS_PER_GROUP

    idx4 = idx_feed.reshape(NW, ngrp, G, CH)
    gathered = _make_sc_gather(NW, ngrp, d_out)(u_rows, idx4)

    g_packed = gathered.reshape(seq, batch // 2, 2 * d_out)
    p = _transpose_out(g_packed, seq, batch, d_out)
    return jnp.transpose(p, (2, 0, 1))


# speedup vs baseline: 2.4869x; 1.0785x over previous
import functools

import jax
import jax.numpy as jnp
from jax import lax
from jax.experimental import pallas as pl
from jax.experimental.pallas import tpu as pltpu
from jax.experimental.pallas import tpu_sc as plsc

CH = 128
G = 4
ROWS_PER_GROUP = CH * G

VB = 8192
VB_SHIFT = 13


@functools.cache
def _make_sc_gather(NW, NGRP, D):
    mesh = plsc.VectorSubcoreMesh(core_axis_name="c", subcore_axis_name="s")
    info = plsc.get_sparse_core_info()
    NC = info.num_cores

    @functools.partial(
        pl.kernel,
        mesh=mesh,
        compiler_params=pltpu.CompilerParams(use_tc_tiling_on_sc=False),
        out_type=jax.ShapeDtypeStruct((NW, NGRP, ROWS_PER_GROUP, D), jnp.float32),
        scratch_types=[
            pltpu.VMEM((NGRP, G, CH), jnp.int32),
            pltpu.VMEM((2, ROWS_PER_GROUP, D), jnp.float32),
            pltpu.SemaphoreType.DMA,
            pltpu.SemaphoreType.DMA,
        ],
    )
    def sc_gather(table_hbm, idx_hbm, out_hbm, idx_v, rows_v, sem0, sem1):
        wid = lax.axis_index("s") * NC + lax.axis_index("c")
        pltpu.sync_copy(idx_hbm.at[wid], idx_v)
        sems = (sem0, sem1)

        def fire(g, b):
            for j in range(G):
                pltpu.async_copy(
                    table_hbm.at[idx_v.at[g, j]],
                    rows_v.at[b, pl.ds(j * CH, CH)],
                    sems[b],
                )

        def drain(b):
            pltpu.make_async_copy(
                table_hbm.at[pl.ds(0, ROWS_PER_GROUP)], rows_v.at[b], sems[b]
            ).wait()

        fire(0, 0)
        fire(1, 1)

        def body(i, carry):
            for b in range(2):
                g = 2 * i + b
                drain(b)
                pltpu.sync_copy(rows_v.at[b], out_hbm.at[wid, g])
                fire(g + 2, b)
            return carry

        lax.fori_loop(0, NGRP // 2 - 1, body, 0)
        for b in range(2):
            g = NGRP - 2 + b
            drain(b)
            pltpu.sync_copy(rows_v.at[b], out_hbm.at[wid, g])

    return sc_gather


def _project_body(t1_ref, t2_ref, w_ref, b2_ref, o_ref):
    w = w_ref[...]
    cn = (((0,), (0,)), ((), ()))
    z1 = lax.dot_general(t1_ref[...], w, cn,
                         preferred_element_type=jnp.float32)
    z2 = lax.dot_general(t2_ref[...], w, cn,
                         preferred_element_type=jnp.float32)
    z = jnp.concatenate([z1, z2], axis=1)
    o_ref[...] = z + b2_ref[...]


def _project_table(table_t, Wt, b2):
    d_in, vocab = table_t.shape
    grid = pl.cdiv(vocab, 2 * VB)
    n2 = grid * VB
    last = pl.cdiv(vocab, VB) - 1
    return pl.pallas_call(
        _project_body,
        grid=(grid,),
        in_specs=[
            pl.BlockSpec((d_in, VB), lambda i: (0, jnp.minimum(2 * i, last))),
            pl.BlockSpec(
                (d_in, VB), lambda i: (0, jnp.minimum(2 * i + 1, last))
            ),
            pl.BlockSpec((d_in, d_in), lambda i: (0, 0)),
            pl.BlockSpec((1, 128), lambda i: (0, 0)),
        ],
        out_specs=pl.BlockSpec((VB, 128), lambda i: (i, 0)),
        out_shape=jax.ShapeDtypeStruct((n2, 128), jnp.float32),
    )(table_t, table_t, Wt, b2)


def _make_unpack_body(half_blocks):
    def _unpack_body(g_ref, o_ref):
        t_all = g_ref[0].T
        half = pl.program_id(1) >= half_blocks
        o_ref[0] = jnp.where(half, t_all[64:], t_all[:64])
    return _unpack_body


def _transpose_out(g_packed, seq, batch, d_out):
    cb = 8192
    half_blocks = (batch // 2) // cb
    return pl.pallas_call(
        _make_unpack_body(half_blocks),
        grid=(seq, 2 * half_blocks),
        in_specs=[
            pl.BlockSpec((1, cb, 128),
                         lambda s, j: (s, lax.rem(j, half_blocks), 0)),
        ],
        out_specs=pl.BlockSpec((1, d_out, cb), lambda s, j: (s, 0, j)),
        out_shape=jax.ShapeDtypeStruct((seq, d_out, batch), jnp.float32),
    )(g_packed)


def kernel(indices, table, W, b):
    batch, seq = indices.shape
    vocab, d = table.shape
    d_out = W.shape[1]
    n_rows = batch * seq

    table_t = jnp.transpose(table)
    idx_t = jnp.transpose(indices.astype(jnp.int32))

    u2 = _project_table(table_t, W, jnp.tile(b, 2).reshape(1, 2 * d_out))
    u_rows = u2.reshape(u2.shape[0] * 2, d_out)

    t_tile = idx_t >> VB_SHIFT
    v_packed = (
        (((t_tile >> 1) << VB_SHIFT) | (idx_t & (VB - 1))) * 2 + (t_tile & 1)
    )
    idx_feed = (
        v_packed.reshape(seq, 2, batch // 2)
        .transpose(0, 2, 1)
        .reshape(seq, batch)
    )

    info = plsc.get_sparse_core_info()
    NW = info.num_cores * info.num_subcores
    per_w = n_rows // NW
    assert per_w * NW == n_rows and per_w % ROWS_PER_GROUP == 0
    ngrp = per_w // ROWS_PER_GROUP

    idx4 = idx_feed.reshape(NW, ngrp, G, CH)
    gathered = _make_sc_gather(NW, ngrp, d_out)(u_rows, idx4)

    g_packed = gathered.reshape(seq, batch // 2, 2 * d_out)
    p = _transpose_out(g_packed, seq, batch, d_out)
    return jnp.transpose(p, (2, 0, 1))
